# baseline (speedup 1.0000x reference)
import functools

import jax
import jax.numpy as jnp
from jax import lax
from jax.experimental import pallas as pl
from jax.experimental.pallas import tpu as pltpu
from jax.experimental.pallas import tpu_sc as plsc

NBUF = 8


@functools.lru_cache(maxsize=None)
def _build(batch: int, seq: int, d: int, n_workers: int):
    rows_per_w = batch // n_workers
    mesh = plsc.VectorSubcoreMesh(core_axis_name="c", subcore_axis_name="s")

    @functools.partial(
        pl.kernel,
        mesh=mesh,
        out_type=jax.ShapeDtypeStruct((batch, seq, d), jnp.float32),
        scratch_types=[
            pltpu.VMEM((rows_per_w, seq), jnp.int32),
            pltpu.VMEM((NBUF, seq, d), jnp.float32),
        ] + [pltpu.SemaphoreType.DMA] * NBUF,
        compiler_params=pltpu.CompilerParams(use_tc_tiling_on_sc=False),
    )
    def k(idx_hbm, weight_hbm, out_hbm, idx_v, rows_v, *gsems):
        nc = plsc.get_sparse_core_info().num_cores
        wid = lax.axis_index("s") * nc + lax.axis_index("c")
        base = wid * rows_per_w
        pltpu.sync_copy(idx_hbm.at[pl.ds(base, rows_per_w)], idx_v)

        for b in range(NBUF):
            pltpu.async_copy(weight_hbm.at[idx_v.at[b]], rows_v.at[b],
                             gsems[b])

        def step(s, carry):
            for b in range(NBUF):
                g = s * NBUF + b
                pltpu.make_async_copy(weight_hbm.at[idx_v.at[g]],
                                      rows_v.at[b], gsems[b]).wait()
                pltpu.sync_copy(rows_v.at[b], out_hbm.at[base + g])
                gn = jnp.minimum(g + NBUF, rows_per_w - 1)
                pltpu.async_copy(weight_hbm.at[idx_v.at[gn]], rows_v.at[b],
                                 gsems[b])
            return carry

        lax.fori_loop(0, rows_per_w // NBUF, step, 0)

        for b in range(NBUF):
            pltpu.make_async_copy(weight_hbm.at[idx_v.at[rows_per_w - 1]],
                                  rows_v.at[b], gsems[b]).wait()

    return k


def kernel(token_ids, weight):
    batch, seq = token_ids.shape
    vocab, d = weight.shape
    info = plsc.get_sparse_core_info()
    n_workers = info.num_cores * info.num_subcores
    return _build(batch, seq, d, n_workers)(token_ids.astype(jnp.int32),
                                            weight)


# speedup vs baseline: 1.0014x; 1.0014x over previous
import functools

import jax
import jax.numpy as jnp
from jax import lax
from jax.experimental import pallas as pl
from jax.experimental.pallas import tpu as pltpu
from jax.experimental.pallas import tpu_sc as plsc

CHUNK = 128
NBUF = 8


@functools.lru_cache(maxsize=None)
def _build(batch: int, seq: int, d: int, n_workers: int):
    b_per_w = batch // n_workers
    n_h = b_per_w // CHUNK
    n_chunks = seq * n_h
    mesh = plsc.VectorSubcoreMesh(core_axis_name="c", subcore_axis_name="s")

    @functools.partial(
        pl.kernel,
        mesh=mesh,
        out_type=jax.ShapeDtypeStruct((batch, seq, d), jnp.float32),
        scratch_types=[
            pltpu.VMEM((seq, b_per_w), jnp.int32),
            pltpu.VMEM((NBUF, CHUNK, d), jnp.float32),
        ] + [pltpu.SemaphoreType.DMA] * NBUF,
        compiler_params=pltpu.CompilerParams(use_tc_tiling_on_sc=False),
    )
    def k(idxt_hbm, weight_hbm, out_hbm, idx_v, rows_v, *gsems):
        nc = plsc.get_sparse_core_info().num_cores
        wid = lax.axis_index("s") * nc + lax.axis_index("c")
        base = wid * b_per_w
        pltpu.sync_copy(idxt_hbm.at[:, pl.ds(base, b_per_w)], idx_v)

        def idx_at(g):
            s, h = g // n_h, g % n_h
            return idx_v.at[s, pl.ds(h * CHUNK, CHUNK)]

        def out_at(g):
            s, h = g // n_h, g % n_h
            return out_hbm.at[pl.ds(base + h * CHUNK, CHUNK), s]

        for b in range(NBUF):
            pltpu.async_copy(weight_hbm.at[idx_at(b)], rows_v.at[b],
                             gsems[b])

        def step(st, carry):
            for b in range(NBUF):
                g = st * NBUF + b
                pltpu.make_async_copy(weight_hbm.at[idx_at(g)],
                                      rows_v.at[b], gsems[b]).wait()
                pltpu.sync_copy(rows_v.at[b], out_at(g))
                gn = jnp.minimum(g + NBUF, n_chunks - 1)
                pltpu.async_copy(weight_hbm.at[idx_at(gn)], rows_v.at[b],
                                 gsems[b])
            return carry

        lax.fori_loop(0, n_chunks // NBUF, step, 0)

        for b in range(NBUF):
            pltpu.make_async_copy(weight_hbm.at[idx_at(n_chunks - 1)],
                                  rows_v.at[b], gsems[b]).wait()

    return k


def kernel(token_ids, weight):
    batch, seq = token_ids.shape
    vocab, d = weight.shape
    info = plsc.get_sparse_core_info()
    n_workers = info.num_cores * info.num_subcores
    idx_t = token_ids.T.astype(jnp.int32)
    return _build(batch, seq, d, n_workers)(idx_t, weight)
